# baseline (speedup 1.0000x reference)
import functools

import jax
import jax.numpy as jnp
from jax import lax
from jax.experimental import pallas as pl
from jax.experimental.pallas import tpu as pltpu
from jax.experimental.pallas import tpu_sc as plsc

M = 64
M_SHIFT = 6
L = 16


def _sc_route_dispatch(x, gi, B, D, G, NB, P):
    info = plsc.get_sparse_core_info()
    NC, NS = info.num_cores, info.num_subcores
    NW = NC * NS
    tpw = B // NW
    nv = tpw // L
    nv_all = B // L
    mesh = plsc.VectorSubcoreMesh(core_axis_name="c", subcore_axis_name="s")

    @functools.partial(
        pl.kernel,
        mesh=mesh,
        out_type=(
            jax.ShapeDtypeStruct((P, D), jnp.float32),
            jax.ShapeDtypeStruct((NB,), jnp.int32),
            jax.ShapeDtypeStruct((B,), jnp.int32),
        ),
        scratch_types=[
            pltpu.VMEM((B,), jnp.int32),
            pltpu.VMEM((tpw, D), jnp.float32),
            pltpu.VMEM((tpw,), jnp.int32),
            pltpu.VMEM((((NB + L - 1) // L) * L,), jnp.int32),
            pltpu.SemaphoreType.DMA,
            pltpu.SemaphoreType.DMA,
        ],
        compiler_params=pltpu.CompilerParams(needs_layout_passes=False),
    )
    def k(x_hbm, gi_hbm, xs_hbm, bg_hbm, slots_hbm, gi_v, rows_v, slot_v,
          bg_v, sem_x, sem_s):
        wid = lax.axis_index("s") * NC + lax.axis_index("c")
        base = wid * tpw
        cp_x = pltpu.async_copy(x_hbm.at[pl.ds(base, tpw)], rows_v, sem_x)
        pltpu.sync_copy(gi_hbm, gi_v)

        zero_s = jnp.int32(0)
        tots = [zero_s] * G
        prevs = [zero_s] * G

        def count_body(i, carry):
            tots, prevs = list(carry[0]), list(carry[1])
            vec = gi_v[pl.ds(i * L, L)]
            before = jnp.where(i < wid * nv, jnp.int32(1), jnp.int32(0))
            for g in range(G):
                pc = jnp.sum((vec == g).astype(jnp.int32))
                tots[g] = tots[g] + pc
                prevs[g] = prevs[g] + pc * before
            return (tuple(tots), tuple(prevs))

        tots, prevs = lax.fori_loop(0, nv_all, count_body,
                                    (tuple(tots), tuple(prevs)))

        nblk = [(tots[g] + (M - 1)) >> M_SHIFT for g in range(G)]
        blk_start = []
        acc = zero_s
        for g in range(G):
            blk_start.append(acc)
            acc = acc + nblk[g]
        blk_cum = [blk_start[g] + nblk[g] for g in range(G)]

        iota = lax.iota(jnp.int32, L)
        zero_v = jnp.zeros((L,), jnp.int32)
        for half in range((NB + L - 1) // L):
            ivec = iota + half * L
            bg = zero_v
            for g in range(G):
                bg = bg + jnp.where(ivec >= blk_cum[g], 1, 0).astype(jnp.int32)
            bg_v[pl.ds(half * L, L)] = jnp.minimum(bg, G - 1)

        @pl.when(wid == 0)
        def _():
            pltpu.sync_copy(bg_v.at[pl.ds(0, NB)], bg_hbm)

        carry = [prevs[g] for g in range(G)]
        for v in range(nv):
            vec = gi_v[pl.ds((wid * nv + v) * L, L)]
            slot = zero_v
            for g in range(G):
                mi = (vec == g).astype(jnp.int32)
                rank = plsc.cumsum(mi) - mi + carry[g]
                slot = slot + mi * (blk_start[g] * M + rank)
                carry[g] = carry[g] + jnp.sum(mi)
            slot_v[pl.ds(v * L, L)] = slot

        pltpu.sync_copy(slot_v, slots_hbm.at[pl.ds(base, tpw)])
        cp_x.wait()
        pltpu.async_copy(rows_v, xs_hbm.at[slot_v], sem_s).wait()

    return k(x, gi)


def _sc_combine(y_sorted, slots, B, P, D):
    info = plsc.get_sparse_core_info()
    NC, NS = info.num_cores, info.num_subcores
    NW = NC * NS
    tpw = B // NW
    mesh = plsc.VectorSubcoreMesh(core_axis_name="c", subcore_axis_name="s")

    @functools.partial(
        pl.kernel,
        mesh=mesh,
        out_type=jax.ShapeDtypeStruct((B, D), jnp.float32),
        scratch_types=[
            pltpu.VMEM((tpw,), jnp.int32),
            pltpu.VMEM((tpw, D), jnp.float32),
            pltpu.SemaphoreType.DMA,
        ],
        compiler_params=pltpu.CompilerParams(needs_layout_passes=False),
    )
    def k(ys_hbm, slots_hbm, y_hbm, slot_v, rows_v, sem):
        wid = lax.axis_index("s") * NC + lax.axis_index("c")
        base = wid * tpw
        pltpu.sync_copy(slots_hbm.at[pl.ds(base, tpw)], slot_v)
        pltpu.async_copy(ys_hbm.at[slot_v], rows_v, sem).wait()
        pltpu.sync_copy(rows_v, y_hbm.at[pl.ds(base, tpw)])

    return k(y_sorted, slots)


def _tc_grouped_matmul(x_sorted, W, b, block_group, NB, D):
    def body(bg_ref, x_ref, w_ref, b_ref, o_ref):
        o_ref[...] = (
            jnp.dot(x_ref[...], w_ref[0], preferred_element_type=jnp.float32)
            + b_ref[0]
        )

    G = W.shape[0]
    grid_spec = pltpu.PrefetchScalarGridSpec(
        num_scalar_prefetch=1,
        grid=(NB,),
        in_specs=[
            pl.BlockSpec((M, D), lambda i, bg: (i, 0)),
            pl.BlockSpec((1, D, D), lambda i, bg: (bg[i], 0, 0)),
            pl.BlockSpec((1, 1, D), lambda i, bg: (bg[i], 0, 0)),
        ],
        out_specs=pl.BlockSpec((M, D), lambda i, bg: (i, 0)),
    )
    return pl.pallas_call(
        body,
        grid_spec=grid_spec,
        out_shape=jax.ShapeDtypeStruct((NB * M, D), jnp.float32),
    )(block_group, x_sorted, W, b.reshape(G, 1, D))


def kernel(x, group_indices, W, b):
    B, D = x.shape
    G = W.shape[0]
    NB = B // M + G
    P = NB * M
    gi = group_indices.astype(jnp.int32)
    x_sorted, block_group, slots = _sc_route_dispatch(x, gi, B, D, G, NB, P)
    y_sorted = _tc_grouped_matmul(x_sorted, W, b, block_group, NB, D)
    return _sc_combine(y_sorted, slots, B, P, D)


# speedup vs baseline: 1.1382x; 1.1382x over previous
import functools

import jax
import jax.numpy as jnp
from jax import lax
from jax.experimental import pallas as pl
from jax.experimental.pallas import tpu as pltpu
from jax.experimental.pallas import tpu_sc as plsc

M = 128
M_SHIFT = 7
L = 16
NCHUNK = 4


def _sc_route_dispatch(x, gi, B, D, G, NB, P):
    info = plsc.get_sparse_core_info()
    NC, NS = info.num_cores, info.num_subcores
    NW = NC * NS
    tpw = B // NW
    nv = tpw // L
    nv_all = B // L
    mesh = plsc.VectorSubcoreMesh(core_axis_name="c", subcore_axis_name="s")

    @functools.partial(
        pl.kernel,
        mesh=mesh,
        out_type=(
            jax.ShapeDtypeStruct((P, D), jnp.float32),
            jax.ShapeDtypeStruct((NB,), jnp.int32),
            jax.ShapeDtypeStruct((B // L, L), jnp.int32),
        ),
        scratch_types=[
            pltpu.VMEM((B,), jnp.int32),
            pltpu.VMEM((tpw, D), jnp.float32),
            pltpu.VMEM((NCHUNK, L), jnp.int32),
            pltpu.VMEM((((NB + L - 1) // L) * L,), jnp.int32),
            pltpu.SemaphoreType.DMA,
            pltpu.SemaphoreType.DMA,
        ],
        compiler_params=pltpu.CompilerParams(needs_layout_passes=False),
    )
    def k(x_hbm, gi_hbm, xs_hbm, bg_hbm, slots_hbm, gi_v, rows_v, slot_v,
          bg_v, sem_x, sem_s):
        wid = lax.axis_index("s") * NC + lax.axis_index("c")
        base = wid * tpw
        cp_x = [
            pltpu.async_copy(
                x_hbm.at[pl.ds(base + c * L, L)],
                rows_v.at[pl.ds(c * L, L)],
                sem_x,
            )
            for c in range(NCHUNK)
        ]
        pltpu.sync_copy(gi_hbm, gi_v)

        zero_s = jnp.int32(0)
        tots = [zero_s] * G
        prevs = [zero_s] * G

        def count_body(i, carry):
            tots, prevs = list(carry[0]), list(carry[1])
            vec = gi_v[pl.ds(i * L, L)]
            before = jnp.where(i < wid * nv, jnp.int32(1), jnp.int32(0))
            for g in range(G):
                pc = jnp.sum((vec == g).astype(jnp.int32))
                tots[g] = tots[g] + pc
                prevs[g] = prevs[g] + pc * before
            return (tuple(tots), tuple(prevs))

        tots, prevs = lax.fori_loop(0, nv_all, count_body,
                                    (tuple(tots), tuple(prevs)))

        nblk = [(tots[g] + (M - 1)) >> M_SHIFT for g in range(G)]
        blk_start = []
        acc = zero_s
        for g in range(G):
            blk_start.append(acc)
            acc = acc + nblk[g]
        blk_cum = [blk_start[g] + nblk[g] for g in range(G)]

        iota = lax.iota(jnp.int32, L)
        zero_v = jnp.zeros((L,), jnp.int32)
        for half in range((NB + L - 1) // L):
            ivec = iota + half * L
            bg = zero_v
            for g in range(G):
                bg = bg + jnp.where(ivec >= blk_cum[g], 1, 0).astype(jnp.int32)
            bg_v[pl.ds(half * L, L)] = jnp.minimum(bg, G - 1)

        @pl.when(wid == 0)
        def _():
            pltpu.sync_copy(bg_v.at[pl.ds(0, NB)], bg_hbm)

        carry = [prevs[g] for g in range(G)]
        for v in range(nv):
            vec = gi_v[pl.ds((wid * nv + v) * L, L)]
            slot = zero_v
            for g in range(G):
                mi = (vec == g).astype(jnp.int32)
                rank = plsc.cumsum(mi) - mi + carry[g]
                slot = slot + mi * (blk_start[g] * M + rank)
                carry[g] = carry[g] + jnp.sum(mi)
            slot_v[v] = slot

        pltpu.sync_copy(slot_v, slots_hbm.at[pl.ds(wid * NCHUNK, NCHUNK)])
        cp_s = []
        for c in range(NCHUNK):
            cp_x[c].wait()
            cp_s.append(
                pltpu.async_copy(
                    rows_v.at[pl.ds(c * L, L)],
                    xs_hbm.at[slot_v.at[c]],
                    sem_s,
                )
            )
        for c in range(NCHUNK):
            cp_s[c].wait()

    return k(x, gi)


def _sc_combine(y_sorted, slots, B, P, D):
    info = plsc.get_sparse_core_info()
    NC, NS = info.num_cores, info.num_subcores
    NW = NC * NS
    tpw = B // NW
    mesh = plsc.VectorSubcoreMesh(core_axis_name="c", subcore_axis_name="s")

    @functools.partial(
        pl.kernel,
        mesh=mesh,
        out_type=jax.ShapeDtypeStruct((B, D), jnp.float32),
        scratch_types=[
            pltpu.VMEM((NCHUNK, L), jnp.int32),
            pltpu.VMEM((tpw, D), jnp.float32),
            pltpu.SemaphoreType.DMA,
            pltpu.SemaphoreType.DMA,
        ],
        compiler_params=pltpu.CompilerParams(needs_layout_passes=False),
    )
    def k(ys_hbm, slots_hbm, y_hbm, slot_v, rows_v, sem_g, sem_w):
        wid = lax.axis_index("s") * NC + lax.axis_index("c")
        base = wid * tpw
        pltpu.sync_copy(slots_hbm.at[pl.ds(wid * NCHUNK, NCHUNK)], slot_v)
        cp_g = [
            pltpu.async_copy(
                ys_hbm.at[slot_v.at[c]],
                rows_v.at[pl.ds(c * L, L)],
                sem_g,
            )
            for c in range(NCHUNK)
        ]
        cp_w = []
        for c in range(NCHUNK):
            cp_g[c].wait()
            cp_w.append(
                pltpu.async_copy(
                    rows_v.at[pl.ds(c * L, L)],
                    y_hbm.at[pl.ds(base + c * L, L)],
                    sem_w,
                )
            )
        for c in range(NCHUNK):
            cp_w[c].wait()

    return k(y_sorted, slots)


def _tc_grouped_matmul(x_sorted, W, b, block_group, NB, D):
    def body(bg_ref, x_ref, w_ref, b_ref, o_ref):
        o_ref[...] = (
            jnp.dot(x_ref[...], w_ref[0], preferred_element_type=jnp.float32)
            + b_ref[0]
        )

    G = W.shape[0]
    grid_spec = pltpu.PrefetchScalarGridSpec(
        num_scalar_prefetch=1,
        grid=(NB,),
        in_specs=[
            pl.BlockSpec((M, D), lambda i, bg: (i, 0)),
            pl.BlockSpec((1, D, D), lambda i, bg: (bg[i], 0, 0)),
            pl.BlockSpec((1, 1, D), lambda i, bg: (bg[i], 0, 0)),
        ],
        out_specs=pl.BlockSpec((M, D), lambda i, bg: (i, 0)),
    )
    return pl.pallas_call(
        body,
        grid_spec=grid_spec,
        out_shape=jax.ShapeDtypeStruct((NB * M, D), jnp.float32),
    )(block_group, x_sorted, W, b.reshape(G, 1, D))


def kernel(x, group_indices, W, b):
    B, D = x.shape
    G = W.shape[0]
    NB = B // M + G
    P = NB * M
    gi = group_indices.astype(jnp.int32)
    x_sorted, block_group, slots = _sc_route_dispatch(x, gi, B, D, G, NB, P)
    y_sorted = _tc_grouped_matmul(x_sorted, W, b, block_group, NB, D)
    return _sc_combine(y_sorted, slots, B, P, D)
